# Initial kernel scaffold; baseline (speedup 1.0000x reference)
import jax
import jax.numpy as jnp
from jax.experimental import pallas as pl


def kernel(x, pos, edge_index, pe, Wm1, bm1, Wm2, bm2, Wp1, bp1, Wp2, bp2, Wu1, bu1, Wu2, bu2, Wq1, bq1, Wq2, bq2):
    raise NotImplementedError("write your pallas kernel here")


# speedup vs baseline: 8.5416x; 8.5416x over previous
import functools

import jax
import jax.numpy as jnp
from jax import lax
from jax.experimental import pallas as pl
from jax.experimental.pallas import tpu as pltpu
from jax.experimental.pallas import tpu_sc as plsc

N = 10000
E = 320000
H = 128

NB = 1000
EB = 2000

NC = 2
NS = 16
NW = NC * NS
CB = 80

EW = E // NW
BCH = EW // CB
ET = E // NS
DCH = ET // CB
NP = 10240
NR = NP // NS

_f32 = jnp.float32
_bf16 = jnp.bfloat16
_mesh = plsc.VectorSubcoreMesh(
    core_axis_name="c", subcore_axis_name="s", num_cores=NC, num_subcores=NS)


def _node_pre_body(x_ref, pe_ref, wmab_ref, wpcd_ref, wuab_ref, wqa_ref,
                   bm_ref, bp_ref, bxu_ref, bpq_ref,
                   src_ref, dst_ref, xu_ref, pq_ref):
    x = x_ref[...]
    pe = pe_ref[...]
    xpe = jnp.concatenate([x, pe], axis=1)
    ab = jnp.dot(xpe, wmab_ref[...], preferred_element_type=_f32)
    cd = jnp.dot(pe, wpcd_ref[...], preferred_element_type=_f32)
    def _b16(v):
        u = lax.bitcast_convert_type(v, jnp.int32)
        return ((u + 0x7FFF + ((u >> 16) & 1)) >> 16) & 0xFFFF

    def _pack(lo, hi):
        return (_b16(hi) << 16) | _b16(lo)

    src_ref[...] = _pack(ab[:, :H] + bm_ref[...], cd[:, :H] + bp_ref[...])
    dst_ref[...] = _pack(ab[:, H:], cd[:, H:])
    xu_ref[...] = jnp.dot(xpe, wuab_ref[...], preferred_element_type=_f32) + bxu_ref[...]
    pq_ref[...] = jnp.dot(pe, wqa_ref[...], preferred_element_type=_f32) + bpq_ref[...]


def _node_pre(x, pe, wmab, wpcd, wuab, wqa, bm, bp, bxu, bpq):
    w = lambda s: pl.BlockSpec(s, lambda i: (0, 0))
    return pl.pallas_call(
        _node_pre_body,
        grid=(N // NB,),
        in_specs=[
            pl.BlockSpec((NB, H), lambda i: (i, 0)),
            pl.BlockSpec((NB, H), lambda i: (i, 0)),
            w((2 * H, 2 * H)), w((H, 2 * H)), w((2 * H, H)), w((H, H)),
            w((1, H)), w((1, H)), w((1, H)), w((1, H)),
        ],
        out_specs=[
            pl.BlockSpec((NB, H), lambda i: (i, 0)),
            pl.BlockSpec((NB, H), lambda i: (i, 0)),
            pl.BlockSpec((NB, H), lambda i: (i, 0)),
            pl.BlockSpec((NB, H), lambda i: (i, 0)),
        ],
        out_shape=[
            jax.ShapeDtypeStruct((N, H), jnp.int32),
            jax.ShapeDtypeStruct((N, H), jnp.int32),
            jax.ShapeDtypeStruct((N, H), _f32),
            jax.ShapeDtypeStruct((N, H), _f32),
        ],
    )(x, pe, wmab, wpcd, wuab, wqa, bm, bp, bxu, bpq)


@functools.partial(
    pl.kernel,
    out_type=jax.ShapeDtypeStruct((E,), _f32),
    mesh=_mesh,
    compiler_params=pltpu.CompilerParams(needs_layout_passes=False),
    scratch_types=[
        pltpu.VMEM((EW,), jnp.int32),
        pltpu.VMEM((EW,), jnp.int32),
        pltpu.VMEM((EW,), _f32),
        pltpu.VMEM((N,), _f32),
        pltpu.VMEM((N,), _f32),
        pltpu.VMEM((N,), _f32),
    ],
)
def _edge_dist2(send_hbm, rec_hbm, px_hbm, py_hbm, pz_hbm, d2_hbm,
                sidx, ridx, d2b, px, py, pz):
    cid = lax.axis_index("c")
    sid = lax.axis_index("s")
    wid = sid * NC + cid
    pltpu.sync_copy(px_hbm, px)
    pltpu.sync_copy(py_hbm, py)
    pltpu.sync_copy(pz_hbm, pz)
    pltpu.sync_copy(send_hbm.at[pl.ds(wid * EW, EW)], sidx)
    pltpu.sync_copy(rec_hbm.at[pl.ds(wid * EW, EW)], ridx)

    def step(i, carry):
        sl = pl.ds(i * 16, 16)
        si = sidx[sl]
        ri = ridx[sl]
        dx = plsc.load_gather(px, [si]) - plsc.load_gather(px, [ri])
        dy = plsc.load_gather(py, [si]) - plsc.load_gather(py, [ri])
        dz = plsc.load_gather(pz, [si]) - plsc.load_gather(pz, [ri])
        d2b[sl] = dx * dx + dy * dy + dz * dz
        return carry

    lax.fori_loop(0, EW // 16, step, 0)
    pltpu.sync_copy(d2b, d2_hbm.at[pl.ds(wid * EW, EW)])


def _make_edge_gather(e0, ne, cb):
    ew = ne // NW
    bch = ew // cb
    assert ew % cb == 0 and cb % 8 == 0 and cb <= 128

    @functools.partial(
        pl.kernel,
        out_type=jax.ShapeDtypeStruct((ne, H), jnp.int32),
        mesh=_mesh,
        compiler_params=pltpu.CompilerParams(needs_layout_passes=False),
        scratch_types=[
            pltpu.VMEM((cb, H), jnp.int32),
            pltpu.VMEM((cb, H), jnp.int32),
            pltpu.VMEM((cb, H), jnp.int32),
            pltpu.VMEM((cb, H), jnp.int32),
            pltpu.VMEM((cb, H), jnp.int32),
            pltpu.VMEM((cb, H), jnp.int32),
            pltpu.VMEM((cb,), jnp.int32),
            pltpu.VMEM((cb,), jnp.int32),
            pltpu.VMEM((cb,), jnp.int32),
            pltpu.VMEM((cb,), jnp.int32),
            pltpu.SemaphoreType.DMA,
            pltpu.SemaphoreType.DMA,
            pltpu.SemaphoreType.DMA,
            pltpu.SemaphoreType.DMA,
        ],
    )
    def gather(src_hbm, dst_hbm, send_hbm, rec_hbm, pre_hbm,
               sb0, sb1, rb0, rb1, ob0, ob1, si0, si1, ri0, ri1,
               gs0, gs1, ws0, ws1):
        cid = lax.axis_index("c")
        sid = lax.axis_index("s")
        wid = sid * NC + cid
        sets = ((sb0, rb0, ob0, si0, ri0, gs0, ws0),
                (sb1, rb1, ob1, si1, ri1, gs1, ws1))

        def issue(c, sb, rb, si, ri, gs):
            base = wid * ew + c * cb
            pltpu.sync_copy(send_hbm.at[pl.ds(e0 + base, cb)], si)
            pltpu.sync_copy(rec_hbm.at[pl.ds(e0 + base, cb)], ri)
            pltpu.async_copy(src_hbm.at[si], sb, gs)
            pltpu.async_copy(dst_hbm.at[ri], rb, gs)

        def visit(c, sb, rb, ob, si, ri, gs, ws):
            pltpu.make_async_copy(src_hbm.at[pl.ds(0, cb)], sb, gs).wait()
            pltpu.make_async_copy(src_hbm.at[pl.ds(0, cb)], rb, gs).wait()

            @pl.when(c >= 2)
            def _():
                pltpu.make_async_copy(ob, pre_hbm.at[pl.ds(0, cb)], ws).wait()

            def addrow(r, carry):
                for j in range(H // 16):
                    sl = pl.ds(j * 16, 16)
                    a = plsc.bitcast(sb[r, sl], _bf16)
                    b = plsc.bitcast(rb[r, sl], _bf16)
                    ob[r, sl] = plsc.bitcast(a + b, jnp.int32)
                return carry

            lax.fori_loop(0, cb, addrow, 0)
            base = wid * ew + c * cb
            pltpu.async_copy(ob, pre_hbm.at[pl.ds(base, cb)], ws)

            @pl.when(c + 2 < bch)
            def _():
                issue(c + 2, sb, rb, si, ri, gs)

        issue(0, *sets[0][:2], *sets[0][3:6])
        issue(1, *sets[1][:2], *sets[1][3:6])

        def pair(g, carry):
            visit(2 * g, *sets[0])
            visit(2 * g + 1, *sets[1])
            return carry

        lax.fori_loop(0, bch // 2, pair, 0)
        if bch % 2:
            visit(jnp.int32(bch - 1), *sets[0])
        pltpu.make_async_copy(ob0, pre_hbm.at[pl.ds(0, cb)], ws0).wait()
        pltpu.make_async_copy(ob1, pre_hbm.at[pl.ds(0, cb)], ws1).wait()

    return gather


def _edge_mlp_body(pre_ref, d2_ref, w5m_ref, w5p_ref, bd2_ref, b2_ref, out_ref):
    dist = jnp.sqrt(d2_ref[...] + 1e-12)
    w = pre_ref[...]
    pm = lax.bitcast_convert_type(w << 16, _f32) + dist * w5m_ref[...]
    pp = lax.bitcast_convert_type(w & jnp.int32(-65536), _f32) + dist * w5p_ref[...]
    h = jnp.concatenate([pm * jax.nn.sigmoid(pm), jnp.tanh(pp)], axis=1)
    q = jnp.dot(h, bd2_ref[...], preferred_element_type=_f32) + b2_ref[...]
    qm = q[:, :H]
    out_ref[...] = jnp.stack([qm * jax.nn.sigmoid(qm), jnp.tanh(q[:, H:])])


def _make_edge_mlp(e0, ne, eb):
    nblk = ne // eb
    ob = e0 // eb
    assert ne % eb == 0 and e0 % eb == 0

    def mlp(pre, d2, w5m, w5p, bd2, b2):
        w = lambda s: pl.BlockSpec(s, lambda i: (0, 0))
        return pl.pallas_call(
            _edge_mlp_body,
            grid=(nblk,),
            in_specs=[
                pl.BlockSpec((eb, H), lambda i: (i, 0)),
                pl.BlockSpec((eb, 1), lambda i: (i + ob, 0)),
                w((1, H)), w((1, H)), w((2 * H, 2 * H)), w((1, 2 * H)),
            ],
            out_specs=pl.BlockSpec((2, eb, H), lambda i: (0, i, 0)),
            out_shape=jax.ShapeDtypeStruct((2, ne, H), _f32),
        )(pre, d2, w5m, w5p, bd2, b2)

    return mlp


def _make_seg_sum(e0, ne, cb):
    et = ne // NS
    dch = et // cb
    assert et % cb == 0

    @functools.partial(
        pl.kernel,
        out_type=jax.ShapeDtypeStruct((2, NP, H), _f32),
        mesh=_mesh,
        compiler_params=pltpu.CompilerParams(needs_layout_passes=False),
        scratch_types=[
            pltpu.VMEM_SHARED((NP, H), _f32),
            pltpu.VMEM((cb, H), _f32),
            pltpu.VMEM((cb, H), _f32),
            pltpu.VMEM((cb,), jnp.int32),
            pltpu.VMEM((cb,), jnp.int32),
            pltpu.SemaphoreType.DMA,
            pltpu.SemaphoreType.DMA,
        ],
    )
    def seg(msgs_hbm, rec_hbm, zeros_hbm, agg_hbm,
            shared, tb0, tb1, ib0, ib1, ls0, ls1):
        cid = lax.axis_index("c")
        sid = lax.axis_index("s")
        pltpu.sync_copy(zeros_hbm.at[pl.ds(sid * NR, NR)],
                        shared.at[pl.ds(sid * NR, NR)])
        plsc.subcore_barrier()
        sets = ((tb0, ib0, ls0), (tb1, ib1, ls1))

        def issue(c, tb, ib, ls):
            base = sid * et + c * cb
            pltpu.async_copy(rec_hbm.at[pl.ds(e0 + base, cb)], ib, ls)
            pltpu.async_copy(msgs_hbm.at[cid, pl.ds(base, cb)], tb, ls)

        def visit(c, tb, ib, ls):
            pltpu.make_async_copy(rec_hbm.at[pl.ds(0, cb)], ib, ls).wait()
            pltpu.make_async_copy(zeros_hbm.at[pl.ds(0, cb)], tb, ls).wait()
            pltpu.sync_copy(tb, shared.at[ib], add=True)

            @pl.when(c + 2 < dch)
            def _():
                issue(c + 2, tb, ib, ls)

        issue(0, *sets[0])
        issue(1, *sets[1])

        def pair(g, carry):
            visit(2 * g, *sets[0])
            visit(2 * g + 1, *sets[1])
            return carry

        lax.fori_loop(0, dch // 2, pair, 0)
        if dch % 2:
            visit(jnp.int32(dch - 1), *sets[0])
        plsc.subcore_barrier()
        pltpu.sync_copy(shared.at[pl.ds(sid * NR, NR)],
                        agg_hbm.at[cid, pl.ds(sid * NR, NR)])

    return seg


def _node_out_body(xu_ref, pq_ref, agg_ref, agg2_ref, bdn_ref, bdo_ref,
                   bu2_ref, bq2_ref, upd_ref, updpe_ref):
    ap = jnp.concatenate([agg_ref[0] + agg2_ref[0],
                          agg_ref[1] + agg2_ref[1]], axis=1)
    t = jnp.dot(ap, bdn_ref[...], preferred_element_type=_f32)
    tm = xu_ref[...] + t[:, :H]
    h = jnp.concatenate(
        [tm * jax.nn.sigmoid(tm), jnp.tanh(pq_ref[...] + t[:, H:])], axis=1)
    o = jnp.dot(h, bdo_ref[...], preferred_element_type=_f32)
    upd_ref[...] = o[:, :H] + bu2_ref[...]
    updpe_ref[...] = jnp.tanh(o[:, H:] + bq2_ref[...])


def _node_out(xu, pq, agg, agg2, bdn, bdo, bu2, bq2):
    w = lambda s: pl.BlockSpec(s, lambda i: (0, 0))
    return pl.pallas_call(
        _node_out_body,
        grid=(N // NB,),
        in_specs=[
            pl.BlockSpec((NB, H), lambda i: (i, 0)),
            pl.BlockSpec((NB, H), lambda i: (i, 0)),
            pl.BlockSpec((2, NB, H), lambda i: (0, i, 0)),
            pl.BlockSpec((2, NB, H), lambda i: (0, i, 0)),
            w((2 * H, 2 * H)), w((2 * H, 2 * H)), w((1, H)), w((1, H)),
        ],
        out_specs=[
            pl.BlockSpec((NB, H), lambda i: (i, 0)),
            pl.BlockSpec((NB, H), lambda i: (i, 0)),
        ],
        out_shape=[
            jax.ShapeDtypeStruct((N, H), _f32),
            jax.ShapeDtypeStruct((N, H), _f32),
        ],
    )(xu, pq, agg, agg2, bdn, bdo, bu2, bq2)


E0 = 163840
E1 = E - E0
_gather_h0 = _make_edge_gather(0, E0, 80)
_gather_h1 = _make_edge_gather(E0, E1, 80)
_mlp_h0 = _make_edge_mlp(0, E0, 1280)
_mlp_h1 = _make_edge_mlp(E0, E1, 1280)
_seg_h0 = _make_seg_sum(0, E0, 128)
_seg_h1 = _make_seg_sum(E0, E1, 80)


def kernel(x, pos, edge_index, pe, Wm1, bm1, Wm2, bm2, Wp1, bp1, Wp2, bp2,
           Wu1, bu1, Wu2, bu2, Wq1, bq1, Wq2, bq2):
    Z = jnp.zeros((H, H), _f32)
    wmab = jnp.concatenate([Wm1[:2 * H], Wm1[2 * H:4 * H]], axis=1)
    wpcd = jnp.concatenate([Wp1[:H], Wp1[H:2 * H]], axis=1)
    wuab = Wu1[:2 * H]
    wqa = Wq1[:H]
    w5m = Wm1[4 * H].reshape(1, H)
    w5p = Wp1[2 * H].reshape(1, H)
    bd2 = jnp.block([[Wm2, Z], [Z, Wp2]])
    b2 = jnp.concatenate([bm2, bp2]).reshape(1, 2 * H)
    bdn = jnp.block([[Wu1[2 * H:3 * H], Z], [Z, Wq1[H:2 * H]]])
    bdo = jnp.block([[Wu2, Z], [Z, Wq2]])

    send = edge_index[0]
    rec = edge_index[1]
    post = pos.T
    px_in, py_in, pz_in = post[0], post[1], post[2]
    zeros = jnp.zeros((NP, H), _f32)

    d2 = _edge_dist2(send, rec, px_in, py_in, pz_in).reshape(E, 1)
    src, dst, xu, pq = _node_pre(x, pe, wmab, wpcd, wuab, wqa,
                                 bm1.reshape(1, H), bp1.reshape(1, H),
                                 bu1.reshape(1, H), bq1.reshape(1, H))
    pre0 = _gather_h0(src, dst, send, rec)
    pre1 = _gather_h1(src, dst, send, rec)
    msgs0 = _mlp_h0(pre0, d2, w5m, w5p, bd2, b2)
    msgs1 = _mlp_h1(pre1, d2, w5m, w5p, bd2, b2)
    agg0 = _seg_h0(msgs0, rec, zeros)
    agg1 = _seg_h1(msgs1, rec, zeros)
    upd, updpe = _node_out(xu, pq, agg0, agg1, bdn, bdo,
                           bu2.reshape(1, H), bq2.reshape(1, H))
    return (upd, updpe)


# speedup vs baseline: 8.8711x; 1.0386x over previous
import functools

import jax
import jax.numpy as jnp
from jax import lax
from jax.experimental import pallas as pl
from jax.experimental.pallas import tpu as pltpu
from jax.experimental.pallas import tpu_sc as plsc

N = 10000
E = 320000
H = 128

NB = 1000
EB = 2000

NC = 2
NS = 16
NW = NC * NS
CB = 80

EW = E // NW
BCH = EW // CB
ET = E // NS
DCH = ET // CB
NP = 10240
NR = NP // NS

_f32 = jnp.float32
_bf16 = jnp.bfloat16
_mesh = plsc.VectorSubcoreMesh(
    core_axis_name="c", subcore_axis_name="s", num_cores=NC, num_subcores=NS)


def _node_pre_body(x_ref, pe_ref, wmab_ref, wpcd_ref, wuab_ref, wqa_ref,
                   bm_ref, bp_ref, bxu_ref, bpq_ref,
                   src_ref, dst_ref, xu_ref, pq_ref):
    x = x_ref[...]
    pe = pe_ref[...]
    xpe = jnp.concatenate([x, pe], axis=1)
    ab = jnp.dot(xpe, wmab_ref[...], preferred_element_type=_f32)
    cd = jnp.dot(pe, wpcd_ref[...], preferred_element_type=_f32)
    def _b16(v):
        u = lax.bitcast_convert_type(v, jnp.int32)
        return ((u + 0x7FFF + ((u >> 16) & 1)) >> 16) & 0xFFFF

    def _pack(lo, hi):
        return (_b16(hi) << 16) | _b16(lo)

    src_ref[...] = _pack(ab[:, :H] + bm_ref[...], cd[:, :H] + bp_ref[...])
    dst_ref[...] = _pack(ab[:, H:], cd[:, H:])
    xu_ref[...] = jnp.dot(xpe, wuab_ref[...], preferred_element_type=_f32) + bxu_ref[...]
    pq_ref[...] = jnp.dot(pe, wqa_ref[...], preferred_element_type=_f32) + bpq_ref[...]


def _node_pre(x, pe, wmab, wpcd, wuab, wqa, bm, bp, bxu, bpq):
    w = lambda s: pl.BlockSpec(s, lambda i: (0, 0))
    return pl.pallas_call(
        _node_pre_body,
        grid=(N // NB,),
        in_specs=[
            pl.BlockSpec((NB, H), lambda i: (i, 0)),
            pl.BlockSpec((NB, H), lambda i: (i, 0)),
            w((2 * H, 2 * H)), w((H, 2 * H)), w((2 * H, H)), w((H, H)),
            w((1, H)), w((1, H)), w((1, H)), w((1, H)),
        ],
        out_specs=[
            pl.BlockSpec((NB, H), lambda i: (i, 0)),
            pl.BlockSpec((NB, H), lambda i: (i, 0)),
            pl.BlockSpec((NB, H), lambda i: (i, 0)),
            pl.BlockSpec((NB, H), lambda i: (i, 0)),
        ],
        out_shape=[
            jax.ShapeDtypeStruct((N, H), jnp.int32),
            jax.ShapeDtypeStruct((N, H), jnp.int32),
            jax.ShapeDtypeStruct((N, H), _f32),
            jax.ShapeDtypeStruct((N, H), _f32),
        ],
    )(x, pe, wmab, wpcd, wuab, wqa, bm, bp, bxu, bpq)


@functools.partial(
    pl.kernel,
    out_type=jax.ShapeDtypeStruct((E,), _f32),
    mesh=_mesh,
    compiler_params=pltpu.CompilerParams(needs_layout_passes=False),
    scratch_types=[
        pltpu.VMEM((EW,), jnp.int32),
        pltpu.VMEM((EW,), jnp.int32),
        pltpu.VMEM((EW,), _f32),
        pltpu.VMEM((N,), _f32),
        pltpu.VMEM((N,), _f32),
        pltpu.VMEM((N,), _f32),
    ],
)
def _edge_dist2(send_hbm, rec_hbm, px_hbm, py_hbm, pz_hbm, d2_hbm,
                sidx, ridx, d2b, px, py, pz):
    cid = lax.axis_index("c")
    sid = lax.axis_index("s")
    wid = sid * NC + cid
    pltpu.sync_copy(px_hbm, px)
    pltpu.sync_copy(py_hbm, py)
    pltpu.sync_copy(pz_hbm, pz)
    pltpu.sync_copy(send_hbm.at[pl.ds(wid * EW, EW)], sidx)
    pltpu.sync_copy(rec_hbm.at[pl.ds(wid * EW, EW)], ridx)

    def step(i, carry):
        sl = pl.ds(i * 16, 16)
        si = sidx[sl]
        ri = ridx[sl]
        dx = plsc.load_gather(px, [si]) - plsc.load_gather(px, [ri])
        dy = plsc.load_gather(py, [si]) - plsc.load_gather(py, [ri])
        dz = plsc.load_gather(pz, [si]) - plsc.load_gather(pz, [ri])
        d2b[sl] = dx * dx + dy * dy + dz * dz
        return carry

    lax.fori_loop(0, EW // 16, step, 0)
    pltpu.sync_copy(d2b, d2_hbm.at[pl.ds(wid * EW, EW)])


def _make_edge_gather(e0, ne, cb):
    ew = ne // NW
    bch = ew // cb
    assert ew % cb == 0 and cb % 8 == 0 and cb <= 128

    @functools.partial(
        pl.kernel,
        out_type=jax.ShapeDtypeStruct((ne, H), jnp.int32),
        mesh=_mesh,
        compiler_params=pltpu.CompilerParams(needs_layout_passes=False),
        scratch_types=[
            pltpu.VMEM((cb, H), jnp.int32),
            pltpu.VMEM((cb, H), jnp.int32),
            pltpu.VMEM((cb, H), jnp.int32),
            pltpu.VMEM((cb, H), jnp.int32),
            pltpu.VMEM((cb, H), jnp.int32),
            pltpu.VMEM((cb, H), jnp.int32),
            pltpu.VMEM((cb,), jnp.int32),
            pltpu.VMEM((cb,), jnp.int32),
            pltpu.VMEM((cb,), jnp.int32),
            pltpu.VMEM((cb,), jnp.int32),
            pltpu.SemaphoreType.DMA,
            pltpu.SemaphoreType.DMA,
            pltpu.SemaphoreType.DMA,
            pltpu.SemaphoreType.DMA,
        ],
    )
    def gather(src_hbm, dst_hbm, send_hbm, rec_hbm, pre_hbm,
               sb0, sb1, rb0, rb1, ob0, ob1, si0, si1, ri0, ri1,
               gs0, gs1, ws0, ws1):
        cid = lax.axis_index("c")
        sid = lax.axis_index("s")
        wid = sid * NC + cid
        sets = ((sb0, rb0, ob0, si0, ri0, gs0, ws0),
                (sb1, rb1, ob1, si1, ri1, gs1, ws1))

        def issue(c, sb, rb, si, ri, gs):
            base = wid * ew + c * cb
            pltpu.sync_copy(send_hbm.at[pl.ds(e0 + base, cb)], si)
            pltpu.sync_copy(rec_hbm.at[pl.ds(e0 + base, cb)], ri)
            pltpu.async_copy(src_hbm.at[si], sb, gs)
            pltpu.async_copy(dst_hbm.at[ri], rb, gs)

        def visit(c, sb, rb, ob, si, ri, gs, ws):
            pltpu.make_async_copy(src_hbm.at[pl.ds(0, cb)], sb, gs).wait()
            pltpu.make_async_copy(src_hbm.at[pl.ds(0, cb)], rb, gs).wait()

            @pl.when(c >= 2)
            def _():
                pltpu.make_async_copy(ob, pre_hbm.at[pl.ds(0, cb)], ws).wait()

            def addrow(r, carry):
                for j in range(H // 16):
                    sl = pl.ds(j * 16, 16)
                    a = plsc.bitcast(sb[r, sl], _bf16)
                    b = plsc.bitcast(rb[r, sl], _bf16)
                    ob[r, sl] = plsc.bitcast(a + b, jnp.int32)
                return carry

            lax.fori_loop(0, cb, addrow, 0)
            base = wid * ew + c * cb
            pltpu.async_copy(ob, pre_hbm.at[pl.ds(base, cb)], ws)

            @pl.when(c + 2 < bch)
            def _():
                issue(c + 2, sb, rb, si, ri, gs)

        issue(0, *sets[0][:2], *sets[0][3:6])
        issue(1, *sets[1][:2], *sets[1][3:6])

        def pair(g, carry):
            visit(2 * g, *sets[0])
            visit(2 * g + 1, *sets[1])
            return carry

        lax.fori_loop(0, bch // 2, pair, 0)
        if bch % 2:
            visit(jnp.int32(bch - 1), *sets[0])
        pltpu.make_async_copy(ob0, pre_hbm.at[pl.ds(0, cb)], ws0).wait()
        pltpu.make_async_copy(ob1, pre_hbm.at[pl.ds(0, cb)], ws1).wait()

    return gather


def _edge_mlp_body(pre_ref, d2_ref, w5m_ref, w5p_ref, bd2_ref, b2_ref, out_ref):
    dist = jnp.sqrt(d2_ref[...] + 1e-12)
    w = pre_ref[...]
    pm = lax.bitcast_convert_type(w << 16, _f32) + dist * w5m_ref[...]
    pp = lax.bitcast_convert_type(w & jnp.int32(-65536), _f32) + dist * w5p_ref[...]
    h = jnp.concatenate([pm * jax.nn.sigmoid(pm), jnp.tanh(pp)], axis=1)
    q = jnp.dot(h, bd2_ref[...], preferred_element_type=_f32) + b2_ref[...]
    qm = q[:, :H]
    out_ref[...] = jnp.stack([qm * jax.nn.sigmoid(qm), jnp.tanh(q[:, H:])])


def _make_edge_mlp(e0, ne, eb):
    nblk = ne // eb
    ob = e0 // eb
    assert ne % eb == 0 and e0 % eb == 0

    def mlp(pre, d2, w5m, w5p, bd2, b2):
        w = lambda s: pl.BlockSpec(s, lambda i: (0, 0))
        return pl.pallas_call(
            _edge_mlp_body,
            grid=(nblk,),
            in_specs=[
                pl.BlockSpec((eb, H), lambda i: (i, 0)),
                pl.BlockSpec((eb, 1), lambda i: (i + ob, 0)),
                w((1, H)), w((1, H)), w((2 * H, 2 * H)), w((1, 2 * H)),
            ],
            out_specs=pl.BlockSpec((2, eb, H), lambda i: (0, i, 0)),
            out_shape=jax.ShapeDtypeStruct((2, ne, H), _f32),
        )(pre, d2, w5m, w5p, bd2, b2)

    return mlp


def _make_seg_sum(e0, ne, cb):
    et = ne // NS
    dch = et // cb
    assert et % cb == 0

    @functools.partial(
        pl.kernel,
        out_type=jax.ShapeDtypeStruct((2, NP, H), _f32),
        mesh=_mesh,
        compiler_params=pltpu.CompilerParams(needs_layout_passes=False),
        scratch_types=[
            pltpu.VMEM_SHARED((NP, H), _f32),
            pltpu.VMEM((cb, H), _f32),
            pltpu.VMEM((cb, H), _f32),
            pltpu.VMEM((cb,), jnp.int32),
            pltpu.VMEM((cb,), jnp.int32),
            pltpu.SemaphoreType.DMA,
            pltpu.SemaphoreType.DMA,
        ],
    )
    def seg(msgs_hbm, rec_hbm, zeros_hbm, agg_hbm,
            shared, tb0, tb1, ib0, ib1, ls0, ls1):
        cid = lax.axis_index("c")
        sid = lax.axis_index("s")
        pltpu.sync_copy(zeros_hbm.at[pl.ds(sid * NR, NR)],
                        shared.at[pl.ds(sid * NR, NR)])
        plsc.subcore_barrier()
        sets = ((tb0, ib0, ls0), (tb1, ib1, ls1))

        def issue(c, tb, ib, ls):
            base = sid * et + c * cb
            pltpu.async_copy(rec_hbm.at[pl.ds(e0 + base, cb)], ib, ls)
            pltpu.async_copy(msgs_hbm.at[cid, pl.ds(base, cb)], tb, ls)

        def visit(c, tb, ib, ls):
            pltpu.make_async_copy(rec_hbm.at[pl.ds(0, cb)], ib, ls).wait()
            pltpu.make_async_copy(zeros_hbm.at[pl.ds(0, cb)], tb, ls).wait()
            pltpu.sync_copy(tb, shared.at[ib], add=True)

            @pl.when(c + 2 < dch)
            def _():
                issue(c + 2, tb, ib, ls)

        issue(0, *sets[0])
        issue(1, *sets[1])

        def pair(g, carry):
            visit(2 * g, *sets[0])
            visit(2 * g + 1, *sets[1])
            return carry

        lax.fori_loop(0, dch // 2, pair, 0)
        if dch % 2:
            visit(jnp.int32(dch - 1), *sets[0])
        plsc.subcore_barrier()
        pltpu.sync_copy(shared.at[pl.ds(sid * NR, NR)],
                        agg_hbm.at[cid, pl.ds(sid * NR, NR)])

    return seg


def _node_out_body(xu_ref, pq_ref, agg_ref, agg2_ref, bdn_ref, bdo_ref,
                   bu2_ref, bq2_ref, upd_ref, updpe_ref):
    ap = jnp.concatenate([agg_ref[0] + agg2_ref[0],
                          agg_ref[1] + agg2_ref[1]], axis=1)
    t = jnp.dot(ap, bdn_ref[...], preferred_element_type=_f32)
    tm = xu_ref[...] + t[:, :H]
    h = jnp.concatenate(
        [tm * jax.nn.sigmoid(tm), jnp.tanh(pq_ref[...] + t[:, H:])], axis=1)
    o = jnp.dot(h, bdo_ref[...], preferred_element_type=_f32)
    upd_ref[...] = o[:, :H] + bu2_ref[...]
    updpe_ref[...] = jnp.tanh(o[:, H:] + bq2_ref[...])


def _node_out(xu, pq, agg, agg2, bdn, bdo, bu2, bq2):
    w = lambda s: pl.BlockSpec(s, lambda i: (0, 0))
    return pl.pallas_call(
        _node_out_body,
        grid=(N // NB,),
        in_specs=[
            pl.BlockSpec((NB, H), lambda i: (i, 0)),
            pl.BlockSpec((NB, H), lambda i: (i, 0)),
            pl.BlockSpec((2, NB, H), lambda i: (0, i, 0)),
            pl.BlockSpec((2, NB, H), lambda i: (0, i, 0)),
            w((2 * H, 2 * H)), w((2 * H, 2 * H)), w((1, H)), w((1, H)),
        ],
        out_specs=[
            pl.BlockSpec((NB, H), lambda i: (i, 0)),
            pl.BlockSpec((NB, H), lambda i: (i, 0)),
        ],
        out_shape=[
            jax.ShapeDtypeStruct((N, H), _f32),
            jax.ShapeDtypeStruct((N, H), _f32),
        ],
    )(xu, pq, agg, agg2, bdn, bdo, bu2, bq2)


E0 = 153600
E1 = E - E0
_gather_h0 = _make_edge_gather(0, E0, 80)
_gather_h1 = _make_edge_gather(E0, E1, 80)
_mlp_h0 = _make_edge_mlp(0, E0, 1600)
_mlp_h1 = _make_edge_mlp(E0, E1, 1600)
_seg_h0 = _make_seg_sum(0, E0, 80)
_seg_h1 = _make_seg_sum(E0, E1, 80)


def kernel(x, pos, edge_index, pe, Wm1, bm1, Wm2, bm2, Wp1, bp1, Wp2, bp2,
           Wu1, bu1, Wu2, bu2, Wq1, bq1, Wq2, bq2):
    Z = jnp.zeros((H, H), _f32)
    wmab = jnp.concatenate([Wm1[:2 * H], Wm1[2 * H:4 * H]], axis=1)
    wpcd = jnp.concatenate([Wp1[:H], Wp1[H:2 * H]], axis=1)
    wuab = Wu1[:2 * H]
    wqa = Wq1[:H]
    w5m = Wm1[4 * H].reshape(1, H)
    w5p = Wp1[2 * H].reshape(1, H)
    bd2 = jnp.block([[Wm2, Z], [Z, Wp2]])
    b2 = jnp.concatenate([bm2, bp2]).reshape(1, 2 * H)
    bdn = jnp.block([[Wu1[2 * H:3 * H], Z], [Z, Wq1[H:2 * H]]])
    bdo = jnp.block([[Wu2, Z], [Z, Wq2]])

    send = edge_index[0]
    rec = edge_index[1]
    post = pos.T
    px_in, py_in, pz_in = post[0], post[1], post[2]
    zeros = jnp.zeros((NP, H), _f32)

    d2 = _edge_dist2(send, rec, px_in, py_in, pz_in).reshape(E, 1)
    src, dst, xu, pq = _node_pre(x, pe, wmab, wpcd, wuab, wqa,
                                 bm1.reshape(1, H), bp1.reshape(1, H),
                                 bu1.reshape(1, H), bq1.reshape(1, H))
    pre0 = _gather_h0(src, dst, send, rec)
    pre1 = _gather_h1(src, dst, send, rec)
    msgs0 = _mlp_h0(pre0, d2, w5m, w5p, bd2, b2)
    msgs1 = _mlp_h1(pre1, d2, w5m, w5p, bd2, b2)
    agg0 = _seg_h0(msgs0, rec, zeros)
    agg1 = _seg_h1(msgs1, rec, zeros)
    upd, updpe = _node_out(xu, pq, agg0, agg1, bdn, bdo,
                           bu2.reshape(1, H), bq2.reshape(1, H))
    return (upd, updpe)
